# baseline (speedup 1.0000x reference)
import dataclasses
import math

import jax
import jax.numpy as jnp
from jax import lax
from jax.experimental import pallas as pl
from jax.experimental.pallas import tpu as pltpu
from jax.experimental.pallas import tpu_sc as plsc

N = 10000
E = 320000
IN = 128
OUT = 128
T = 3
R = 4
H = 8
DK = 16
ND = 640

NB = 10
BN = N // NB
NW = 32
EPW = 10240
EP = NW * EPW
CH = 32
NCHUNK = EPW // CH
NA = N + 16
RPT = 624
NEG_INF = -3.0e38


def _sc_compiler_params():
    cp = pltpu.CompilerParams()
    if "needs_layout_passes" in pltpu.CompilerParams.__dataclass_fields__:
        cp = dataclasses.replace(cp, needs_layout_passes=False)
    return cp


def _prep_body(x_ref, nt_ref, wk_ref, bk_ref, wq_ref, bq_ref, wv_ref, bv_ref,
               batt_ref, bmsg_ref, pe2_ref, q_ref, kt_ref, vt_ref, mq_ref,
               mk_ref):
    x = x_ref[...]
    nt = nt_ref[...]

    def typed(w_ref, b_ref):
        acc = jnp.zeros((BN, OUT), jnp.float32)
        for t in range(T):
            y = lax.dot_general(x, w_ref[t], (((1,), (0,)), ((), ())),
                                precision=lax.Precision.HIGHEST,
                                preferred_element_type=jnp.float32)
            y = y + b_ref[t][None, :]
            acc = jnp.where(nt == t, y, acc)
        return acc

    def head_sumsq(z):
        return lax.dot_general(z * z, pe2_ref[...], (((1,), (0,)), ((), ())),
                               precision=lax.Precision.HIGHEST,
                               preferred_element_type=jnp.float32)

    k = typed(wk_ref, bk_ref)
    q = typed(wq_ref, bq_ref)
    v = typed(wv_ref, bv_ref)
    q_ref[...] = q
    mq_ref[...] = jnp.max(head_sumsq(q), axis=0, keepdims=True)[None]
    mk = jnp.zeros((BN, H), jnp.float32)
    for r in range(R):
        kt = lax.dot_general(k, batt_ref[r], (((1,), (0,)), ((), ())),
                             precision=lax.Precision.HIGHEST,
                             preferred_element_type=jnp.float32)
        kt_ref[r] = kt
        mk = jnp.maximum(mk, head_sumsq(kt))
        vt_ref[r] = lax.dot_general(v, bmsg_ref[r], (((1,), (0,)), ((), ())),
                                    precision=lax.Precision.HIGHEST,
                                    preferred_element_type=jnp.float32)
    mk_ref[...] = jnp.max(mk, axis=0, keepdims=True)[None]


def _prep(x, nt2, Wk, bk, Wq, bq, Wv, bv, Batt, Bmsg, pe2):
    full = lambda s: pl.BlockSpec(s, lambda i: tuple(0 for _ in s))
    return pl.pallas_call(
        _prep_body,
        grid=(NB,),
        in_specs=[
            pl.BlockSpec((BN, IN), lambda i: (i, 0)),
            pl.BlockSpec((BN, 1), lambda i: (i, 0)),
            full((T, IN, OUT)), full((T, OUT)),
            full((T, IN, OUT)), full((T, OUT)),
            full((T, IN, OUT)), full((T, OUT)),
            full((R, OUT, OUT)), full((R, OUT, OUT)),
            full((IN, H)),
        ],
        out_specs=[
            pl.BlockSpec((BN, OUT), lambda i: (i, 0)),
            pl.BlockSpec((R, BN, OUT), lambda i: (0, i, 0)),
            pl.BlockSpec((R, BN, OUT), lambda i: (0, i, 0)),
            pl.BlockSpec((1, 1, H), lambda i: (i, 0, 0)),
            pl.BlockSpec((1, 1, H), lambda i: (i, 0, 0)),
        ],
        out_shape=[
            jax.ShapeDtypeStruct((NA, OUT), jnp.float32),
            jax.ShapeDtypeStruct((R, N, OUT), jnp.float32),
            jax.ShapeDtypeStruct((R, N, OUT), jnp.float32),
            jax.ShapeDtypeStruct((NB, 1, H), jnp.float32),
            jax.ShapeDtypeStruct((NB, 1, H), jnp.float32),
        ],
    )(x, nt2, Wk, bk, Wq, bq, Wv, bv, Batt, Bmsg, pe2)


def _edge_body(q_hbm, kt_hbm, vt_hbm, src_hbm, dst_hbm, et_hbm, m_hbm,
               zv_hbm, zd_hbm,
               pv_hbm, pd_hbm,
               srcb, etb, idxb, dstc, dsts, ddvs, qb, ktb, vtb, msgb, denb,
               mbuf, isem, gsem, ssem,
               aggv, aggd):
    c = lax.axis_index("c")
    s = lax.axis_index("s")
    wid = s * 2 + c
    base0 = pl.multiple_of(wid * EPW, 8)
    iota = lax.iota(jnp.int32, DK)
    zvec = jnp.zeros((DK,), jnp.float32)

    zoff = pl.multiple_of(s * 632, 8)

    @pl.when(s < 15)
    def _zero_a():
        pltpu.sync_copy(zv_hbm.at[pl.ds(zoff, 632)], aggv.at[pl.ds(zoff, 632)])

    @pl.when(s == 15)
    def _zero_b():
        pltpu.sync_copy(zv_hbm.at[pl.ds(15 * 632, NA - 15 * 632)],
                        aggv.at[pl.ds(15 * 632, NA - 15 * 632)])

    doff = pl.multiple_of(s * (ND // 16), 8)
    pltpu.sync_copy(zd_hbm.at[pl.ds(doff, ND // 16)],
                    aggd.at[pl.ds(doff, ND // 16)])

    pltpu.sync_copy(m_hbm, mbuf)

    @pl.loop(0, CH)
    def _zd(e):
        for p in range(2):
            for j in range(OUT // DK):
                denb[p][e, pl.ds(j * DK, DK)] = zvec

    plsc.subcore_barrier()
    mvals = mbuf[...]

    def idx_load(j, p):
        off = pl.multiple_of(base0 + j * CH, 8)
        pltpu.async_copy(src_hbm.at[pl.ds(off, CH)], srcb[p], isem[p])
        pltpu.async_copy(dst_hbm.at[pl.ds(off, CH)], dstc[p], isem[p])
        pltpu.async_copy(et_hbm.at[pl.ds(off, CH)], etb[p], isem[p])

    def idx_wait(j, p):
        off = pl.multiple_of(base0 + j * CH, 8)
        pltpu.make_async_copy(src_hbm.at[pl.ds(off, CH)], srcb[p],
                              isem[p]).wait()
        pltpu.make_async_copy(dst_hbm.at[pl.ds(off, CH)], dstc[p],
                              isem[p]).wait()
        pltpu.make_async_copy(et_hbm.at[pl.ds(off, CH)], etb[p],
                              isem[p]).wait()

    def idx_compute(p):
        @pl.loop(0, CH // DK)
        def _mk(g):
            sl = pl.ds(g * DK, DK)
            idxb[p][sl] = etb[p][sl] * N + srcb[p][sl]

    def gather_start(p):
        pltpu.async_copy(q_hbm.at[dstc[p]], qb[p], gsem[p])
        pltpu.async_copy(kt_hbm.at[idxb[p]], ktb[p], gsem[p])
        pltpu.async_copy(vt_hbm.at[idxb[p]], vtb[p], gsem[p])

    def gather_wait(p):
        pltpu.make_async_copy(q_hbm.at[dstc[p]], qb[p], gsem[p]).wait()
        pltpu.make_async_copy(kt_hbm.at[idxb[p]], ktb[p], gsem[p]).wait()
        pltpu.make_async_copy(vt_hbm.at[idxb[p]], vtb[p], gsem[p]).wait()

    def scatter_start(p):
        @pl.loop(0, CH // DK)
        def _cp(g):
            sl = pl.ds(g * DK, DK)
            d = dstc[p][sl]
            dsts[p][sl] = d
            ddvs[p][sl] = lax.shift_right_logical(d, 4)

        pltpu.async_copy(msgb[p], aggv.at[dsts[p]], ssem[p], add=True)
        pltpu.async_copy(denb[p], aggd.at[ddvs[p]], ssem[p], add=True)

    def scatter_wait_and_rezero(p):
        pltpu.make_async_copy(msgb[p], aggv.at[dsts[p]], ssem[p]).wait()
        pltpu.make_async_copy(denb[p], aggd.at[ddvs[p]], ssem[p]).wait()

        @pl.loop(0, CH // DK)
        def _rz(g):
            dstv = dsts[p][pl.ds(g * DK, DK)]
            for i in range(DK):
                e = g * DK + i
                col0 = (dstv[i] & 15) * H
                cols = jnp.full((DK,), col0, jnp.int32) + iota
                plsc.store_scatter(denb[p],
                                   [jnp.full((DK,), e, jnp.int32), cols],
                                   zvec, mask=iota < H)

    def compute(p):
        @pl.loop(0, CH // DK)
        def _grp(g):
            dstv = dstc[p][pl.ds(g * DK, DK)]
            for i in range(DK):
                e = g * DK + i
                d_e = dstv[i]
                col0 = (d_e & 15) * H
                cols = jnp.full((DK,), col0, jnp.int32) + iota
                av = jnp.zeros((DK,), jnp.float32)
                for h in range(H):
                    hsl = pl.ds(h * DK, DK)
                    s_h = jnp.sum(qb[p][e, hsl] * ktb[p][e, hsl])
                    av = jnp.where(iota == h, s_h, av)
                ae = jnp.exp(av - mvals)
                plsc.store_scatter(denb[p],
                                   [jnp.full((DK,), e, jnp.int32), cols],
                                   ae, mask=iota < H)
                for h in range(H):
                    hsl = pl.ds(h * DK, DK)
                    msgb[p][e, hsl] = vtb[p][e, hsl] * ae[h]

    idx_load(0, 0)
    idx_wait(0, 0)
    idx_compute(0)
    gather_start(0)
    idx_load(1, 1)

    @pl.loop(0, NCHUNK // 2)
    def _piped(t):
        j0 = t * 2
        for p in range(2):
            j = j0 + p
            pn = 1 - p

            @pl.when(j + 1 < NCHUNK)
            def _a():
                idx_wait(j + 1, pn)

            @pl.when(j >= 2)
            def _b():
                scatter_wait_and_rezero(p)

            @pl.when(j + 1 < NCHUNK)
            def _c():
                idx_compute(pn)
                gather_wait(p)
                gather_start(pn)

            @pl.when(j + 1 >= NCHUNK)
            def _c2():
                gather_wait(p)

            compute(p)
            scatter_start(p)

            @pl.when(j + 2 < NCHUNK)
            def _d():
                idx_load(j + 2, p)

    scatter_wait_and_rezero(0)
    scatter_wait_and_rezero(1)

    plsc.subcore_barrier()
    off = pl.multiple_of(s * RPT, 8)
    pltpu.sync_copy(aggv.at[pl.ds(off, RPT)], pv_hbm.at[c, pl.ds(off, RPT)])

    @pl.when(s == 15)
    def _tail():
        toff = 16 * RPT
        pltpu.sync_copy(aggv.at[pl.ds(toff, N - 16 * RPT)],
                        pv_hbm.at[c, pl.ds(toff, N - 16 * RPT)])

    @pl.when(s == 1)
    def _dtail():
        pltpu.sync_copy(aggd, pd_hbm.at[c])


def _edge(q_tab, kt_tab, vt_tab, src, dst, et, mvec, zv, zd):
    mesh = plsc.VectorSubcoreMesh(core_axis_name="c", subcore_axis_name="s")
    ii = pltpu.VMEM((CH,), jnp.int32)
    ff = pltpu.VMEM((CH, OUT), jnp.float32)
    kern = pl.kernel(
        _edge_body,
        out_type=[
            jax.ShapeDtypeStruct((2, N, OUT), jnp.float32),
            jax.ShapeDtypeStruct((2, ND, OUT), jnp.float32),
        ],
        mesh=mesh,
        compiler_params=_sc_compiler_params(),
        scratch_types=[
            [ii] * 2, [ii] * 2,
            [ii] * 2, [ii] * 2, [ii] * 2, [ii] * 2,
            [ff] * 2, [ff] * 2, [ff] * 2, [ff] * 2, [ff] * 2,
            pltpu.VMEM((DK,), jnp.float32),
            [pltpu.SemaphoreType.DMA] * 2,
            [pltpu.SemaphoreType.DMA] * 2,
            [pltpu.SemaphoreType.DMA] * 2,
            pltpu.VMEM_SHARED((NA, OUT), jnp.float32),
            pltpu.VMEM_SHARED((ND, OUT), jnp.float32),
        ],
    )
    return kern(q_tab, kt_tab, vt_tab, src, dst, et, mvec, zv, zd)


def _update_body(pv_ref, pd_ref, x_ref, nt_ref, wa_ref, ba_ref,
                 gam_ref, bet_ref, sig_ref, pe_ref, o_ref):
    num = pv_ref[0] + pv_ref[1]
    den8 = pd_ref[0] + pd_ref[1]
    den = lax.dot_general(den8, pe_ref[...], (((1,), (0,)), ((), ())),
                          precision=lax.Precision.HIGHEST,
                          preferred_element_type=jnp.float32)
    den = jnp.where(den <= 0.0, 1.0, den)
    aggr = num / den
    g = 0.5 * aggr * (1.0 + lax.erf(aggr * (1.0 / math.sqrt(2.0))))

    nt = nt_ref[...]
    x = x_ref[...]
    trans = jnp.zeros((BN, OUT), jnp.float32)
    a = jnp.zeros((BN, 1), jnp.float32)
    gam = jnp.zeros((BN, OUT), jnp.float32)
    bet = jnp.zeros((BN, OUT), jnp.float32)
    for t in range(T):
        y = lax.dot_general(g, wa_ref[t], (((1,), (0,)), ((), ())),
                            precision=lax.Precision.HIGHEST,
                            preferred_element_type=jnp.float32)
        y = y + ba_ref[t][None, :]
        mask = nt == t
        trans = jnp.where(mask, y, trans)
        a = jnp.where(mask, sig_ref[0, t], a)
        gam = jnp.where(mask, gam_ref[t][None, :], gam)
        bet = jnp.where(mask, bet_ref[t][None, :], bet)

    res = trans * a + x * (1.0 - a)
    mu = jnp.mean(res, axis=1, keepdims=True)
    var = jnp.mean((res - mu) ** 2, axis=1, keepdims=True)
    normed = (res - mu) / jnp.sqrt(var + 1e-5)
    o_ref[...] = normed * gam + bet


def _update(pv, pd8, x, nt2, Wa, ba, gamma, beta, sig, pe):
    full = lambda s: pl.BlockSpec(s, lambda i: tuple(0 for _ in s))
    return pl.pallas_call(
        _update_body,
        grid=(NB,),
        in_specs=[
            pl.BlockSpec((2, BN, OUT), lambda i: (0, i, 0)),
            pl.BlockSpec((2, BN, H), lambda i: (0, i, 0)),
            pl.BlockSpec((BN, IN), lambda i: (i, 0)),
            pl.BlockSpec((BN, 1), lambda i: (i, 0)),
            full((T, OUT, OUT)), full((T, OUT)),
            full((T, OUT)), full((T, OUT)),
            full((1, T)), full((H, OUT)),
        ],
        out_specs=pl.BlockSpec((BN, OUT), lambda i: (i, 0)),
        out_shape=jax.ShapeDtypeStruct((N, OUT), jnp.float32),
    )(pv, pd8, x, nt2, Wa, ba, gamma, beta, sig, pe)


def kernel(meta_xs, node_type, edge_index, edge_type, Wk, bk, Wq, bq, Wv, bv,
           Wa, ba, ln_gamma, ln_beta, relation_pri, relation_att, relation_msg,
           skip):
    x = meta_xs
    nt2 = node_type.reshape(N, 1)
    src = edge_index[0]
    dst = edge_index[1]
    et = edge_type

    eyeH = jnp.eye(H, dtype=jnp.float32)
    att_s = relation_att * (relation_pri / math.sqrt(DK))[:, :, None, None]
    Batt = jnp.einsum('rhdf,hg->rhdgf', att_s, eyeH).reshape(R, OUT, OUT)
    Bmsg = jnp.einsum('rhdf,hg->rhdgf', relation_msg, eyeH).reshape(R, OUT, OUT)

    lane_head0 = jnp.arange(IN, dtype=jnp.int32) // DK
    pe2 = (lane_head0[:, None] == jnp.arange(H, dtype=jnp.int32)[None, :]
           ).astype(jnp.float32)

    q_tab, kt_tab, vt_tab, mq, mk = _prep(x, nt2, Wk, bk, Wq, bq, Wv, bv,
                                          Batt, Bmsg, pe2)
    kt_tab = kt_tab.reshape(R * N, OUT)
    vt_tab = vt_tab.reshape(R * N, OUT)

    mb = jnp.sqrt(jnp.max(mq, axis=0)[0]) * jnp.sqrt(jnp.max(mk, axis=0)[0])
    mvec = jnp.concatenate([mb, jnp.zeros((DK - H,), jnp.float32)])

    pad_n = EP - E
    pr = jnp.arange(pad_n, dtype=jnp.int32)
    src_p = jnp.concatenate([src, pr % N])
    dst_p = jnp.concatenate([dst, N + (pr % 16)])
    et_p = jnp.concatenate([et, pr % R])

    zv = jnp.zeros((NA, OUT), jnp.float32)
    zd = jnp.zeros((ND, OUT), jnp.float32)
    pv, pd = _edge(q_tab, kt_tab, vt_tab, src_p, dst_p, et_p, mvec, zv, zd)

    pd8 = pd.reshape(2, ND * OUT // H, H)[:, :N, :]

    sig = jax.nn.sigmoid(skip).reshape(1, T)
    lane_head = jnp.arange(OUT, dtype=jnp.int32) // DK
    pe = (lane_head[None, :] == jnp.arange(H, dtype=jnp.int32)[:, None]
          ).astype(jnp.float32)

    return _update(pv, pd8, x, nt2, Wa, ba, ln_gamma, ln_beta, sig, pe)


# speedup vs baseline: 1.0711x; 1.0711x over previous
import dataclasses
import math

import jax
import jax.numpy as jnp
from jax import lax
from jax.experimental import pallas as pl
from jax.experimental.pallas import tpu as pltpu
from jax.experimental.pallas import tpu_sc as plsc

N = 10000
E = 320000
IN = 128
OUT = 128
T = 3
R = 4
H = 8
DK = 16
ND = 640

NB = 10
BN = N // NB
NW = 32
EPW = 10240
EP = NW * EPW
CH = 32
NCHUNK = EPW // CH
NA = N + 16
RPT = 624
NEG_INF = -3.0e38


def _sc_compiler_params():
    cp = pltpu.CompilerParams()
    if "needs_layout_passes" in pltpu.CompilerParams.__dataclass_fields__:
        cp = dataclasses.replace(cp, needs_layout_passes=False)
    return cp


def _prep_body(x_ref, nt_ref, wk_ref, bk_ref, wq_ref, bq_ref, wv_ref, bv_ref,
               batt_ref, bmsg_ref, pe2_ref, q_ref, kt_ref, vt_ref, mq_ref,
               mk_ref):
    x = x_ref[...]
    nt = nt_ref[...]

    def typed(w_ref, b_ref):
        acc = jnp.zeros((BN, OUT), jnp.float32)
        for t in range(T):
            y = lax.dot_general(x, w_ref[t], (((1,), (0,)), ((), ())),
                                precision=lax.Precision.HIGHEST,
                                preferred_element_type=jnp.float32)
            y = y + b_ref[t][None, :]
            acc = jnp.where(nt == t, y, acc)
        return acc

    def head_sumsq(z):
        return lax.dot_general(z * z, pe2_ref[...], (((1,), (0,)), ((), ())),
                               precision=lax.Precision.HIGHEST,
                               preferred_element_type=jnp.float32)

    k = typed(wk_ref, bk_ref)
    q = typed(wq_ref, bq_ref)
    v = typed(wv_ref, bv_ref)
    q_ref[...] = q
    mq_ref[...] = jnp.max(head_sumsq(q), axis=0, keepdims=True)[None]
    mk = jnp.zeros((BN, H), jnp.float32)
    for r in range(R):
        kt = lax.dot_general(k, batt_ref[r], (((1,), (0,)), ((), ())),
                             precision=lax.Precision.HIGHEST,
                             preferred_element_type=jnp.float32)
        kt_ref[r] = kt
        mk = jnp.maximum(mk, head_sumsq(kt))
        vt_ref[r] = lax.dot_general(v, bmsg_ref[r], (((1,), (0,)), ((), ())),
                                    precision=lax.Precision.HIGHEST,
                                    preferred_element_type=jnp.float32)
    mk_ref[...] = jnp.max(mk, axis=0, keepdims=True)[None]


def _prep(x, nt2, Wk, bk, Wq, bq, Wv, bv, Batt, Bmsg, pe2):
    full = lambda s: pl.BlockSpec(s, lambda i: tuple(0 for _ in s))
    return pl.pallas_call(
        _prep_body,
        grid=(NB,),
        in_specs=[
            pl.BlockSpec((BN, IN), lambda i: (i, 0)),
            pl.BlockSpec((BN, 1), lambda i: (i, 0)),
            full((T, IN, OUT)), full((T, OUT)),
            full((T, IN, OUT)), full((T, OUT)),
            full((T, IN, OUT)), full((T, OUT)),
            full((R, OUT, OUT)), full((R, OUT, OUT)),
            full((IN, H)),
        ],
        out_specs=[
            pl.BlockSpec((BN, OUT), lambda i: (i, 0)),
            pl.BlockSpec((R, BN, OUT), lambda i: (0, i, 0)),
            pl.BlockSpec((R, BN, OUT), lambda i: (0, i, 0)),
            pl.BlockSpec((1, 1, H), lambda i: (i, 0, 0)),
            pl.BlockSpec((1, 1, H), lambda i: (i, 0, 0)),
        ],
        out_shape=[
            jax.ShapeDtypeStruct((NA, OUT), jnp.float32),
            jax.ShapeDtypeStruct((R, N, OUT), jnp.float32),
            jax.ShapeDtypeStruct((R, N, OUT), jnp.float32),
            jax.ShapeDtypeStruct((NB, 1, H), jnp.float32),
            jax.ShapeDtypeStruct((NB, 1, H), jnp.float32),
        ],
    )(x, nt2, Wk, bk, Wq, bq, Wv, bv, Batt, Bmsg, pe2)


def _edge_body(q_hbm, kt_hbm, vt_hbm, pk_hbm, m_hbm, zv_hbm, zd_hbm,
               pv_hbm, pd_hbm,
               eb, idxb, dstc, dsts, ddvs, qb, ktb, vtb, msgb, denb,
               mbuf, isem, gsem, ssem,
               aggv, aggd):
    c = lax.axis_index("c")
    s = lax.axis_index("s")
    wid = s * 2 + c
    base0 = pl.multiple_of(wid * NCHUNK * (3 * CH), 8)
    iota = lax.iota(jnp.int32, DK)
    zvec = jnp.zeros((DK,), jnp.float32)

    zoff = pl.multiple_of(s * 632, 8)

    @pl.when(s < 15)
    def _zero_a():
        pltpu.sync_copy(zv_hbm.at[pl.ds(zoff, 632)], aggv.at[pl.ds(zoff, 632)])

    @pl.when(s == 15)
    def _zero_b():
        pltpu.sync_copy(zv_hbm.at[pl.ds(15 * 632, NA - 15 * 632)],
                        aggv.at[pl.ds(15 * 632, NA - 15 * 632)])

    doff = pl.multiple_of(s * (ND // 16), 8)
    pltpu.sync_copy(zd_hbm.at[pl.ds(doff, ND // 16)],
                    aggd.at[pl.ds(doff, ND // 16)])

    pltpu.sync_copy(m_hbm, mbuf)

    @pl.loop(0, CH)
    def _zd(e):
        for p in range(2):
            for j in range(OUT // DK):
                denb[p][e, pl.ds(j * DK, DK)] = zvec

    plsc.subcore_barrier()
    mvals = mbuf[...]

    def idx_load(j, p):
        off = pl.multiple_of(base0 + j * (3 * CH), 8)
        pltpu.async_copy(pk_hbm.at[pl.ds(off, 3 * CH)], eb[p], isem[p])

    def idx_wait(j, p):
        off = pl.multiple_of(base0 + j * (3 * CH), 8)
        pltpu.make_async_copy(pk_hbm.at[pl.ds(off, 3 * CH)], eb[p],
                              isem[p]).wait()

    def idx_compute(p):
        @pl.loop(0, CH // DK)
        def _mk(g):
            sl = pl.ds(g * DK, DK)
            sl1 = pl.ds(CH + g * DK, DK)
            sl2 = pl.ds(2 * CH + g * DK, DK)
            idxb[p][sl] = eb[p][sl2] * N + eb[p][sl]
            dstc[p][sl] = eb[p][sl1]

    def gather_start(p):
        pltpu.async_copy(q_hbm.at[dstc[p]], qb[p], gsem[p])
        pltpu.async_copy(kt_hbm.at[idxb[p]], ktb[p], gsem[p])
        pltpu.async_copy(vt_hbm.at[idxb[p]], vtb[p], gsem[p])

    def gather_wait(p):
        pltpu.make_async_copy(q_hbm.at[dstc[p]], qb[p], gsem[p]).wait()
        pltpu.make_async_copy(kt_hbm.at[idxb[p]], ktb[p], gsem[p]).wait()
        pltpu.make_async_copy(vt_hbm.at[idxb[p]], vtb[p], gsem[p]).wait()

    def scatter_start(p):
        @pl.loop(0, CH // DK)
        def _cp(g):
            sl = pl.ds(g * DK, DK)
            d = dstc[p][sl]
            dsts[p][sl] = d
            ddvs[p][sl] = lax.shift_right_logical(d, 4)

        pltpu.async_copy(msgb[p], aggv.at[dsts[p]], ssem[p], add=True)
        pltpu.async_copy(denb[p], aggd.at[ddvs[p]], ssem[p], add=True)

    def scatter_wait_and_rezero(p):
        pltpu.make_async_copy(msgb[p], aggv.at[dsts[p]], ssem[p]).wait()
        pltpu.make_async_copy(denb[p], aggd.at[ddvs[p]], ssem[p]).wait()

        @pl.loop(0, CH // DK)
        def _rz(g):
            dstv = dsts[p][pl.ds(g * DK, DK)]
            for i in range(DK):
                e = g * DK + i
                col0 = (dstv[i] & 15) * H
                cols = jnp.full((DK,), col0, jnp.int32) + iota
                plsc.store_scatter(denb[p],
                                   [jnp.full((DK,), e, jnp.int32), cols],
                                   zvec, mask=iota < H)

    def compute(p):
        @pl.loop(0, CH // DK)
        def _grp(g):
            dstv = dstc[p][pl.ds(g * DK, DK)]
            for i in range(DK):
                e = g * DK + i
                d_e = dstv[i]
                col0 = (d_e & 15) * H
                cols = jnp.full((DK,), col0, jnp.int32) + iota
                av = jnp.zeros((DK,), jnp.float32)
                for h in range(H):
                    hsl = pl.ds(h * DK, DK)
                    s_h = jnp.sum(qb[p][e, hsl] * ktb[p][e, hsl])
                    av = jnp.where(iota == h, s_h, av)
                ae = jnp.exp(av - mvals)
                plsc.store_scatter(denb[p],
                                   [jnp.full((DK,), e, jnp.int32), cols],
                                   ae, mask=iota < H)
                for h in range(H):
                    hsl = pl.ds(h * DK, DK)
                    msgb[p][e, hsl] = vtb[p][e, hsl] * ae[h]

    idx_load(0, 0)
    idx_wait(0, 0)
    idx_compute(0)
    gather_start(0)
    idx_load(1, 1)

    @pl.loop(0, NCHUNK // 2)
    def _piped(t):
        j0 = t * 2
        for p in range(2):
            j = j0 + p
            pn = 1 - p

            @pl.when(j + 1 < NCHUNK)
            def _a():
                idx_wait(j + 1, pn)

            @pl.when(j >= 2)
            def _b():
                scatter_wait_and_rezero(p)

            @pl.when(j + 1 < NCHUNK)
            def _c():
                idx_compute(pn)
                gather_wait(p)
                gather_start(pn)

            @pl.when(j + 1 >= NCHUNK)
            def _c2():
                gather_wait(p)

            @pl.when(j + 2 < NCHUNK)
            def _d():
                idx_load(j + 2, p)

            compute(p)
            scatter_start(p)

    scatter_wait_and_rezero(0)
    scatter_wait_and_rezero(1)

    plsc.subcore_barrier()
    off = pl.multiple_of(s * RPT, 8)
    pltpu.sync_copy(aggv.at[pl.ds(off, RPT)], pv_hbm.at[c, pl.ds(off, RPT)])

    @pl.when(s == 15)
    def _tail():
        toff = 16 * RPT
        pltpu.sync_copy(aggv.at[pl.ds(toff, N - 16 * RPT)],
                        pv_hbm.at[c, pl.ds(toff, N - 16 * RPT)])

    @pl.when(s == 1)
    def _dtail():
        pltpu.sync_copy(aggd, pd_hbm.at[c])


def _edge(q_tab, kt_tab, vt_tab, packed, mvec, zv, zd):
    mesh = plsc.VectorSubcoreMesh(core_axis_name="c", subcore_axis_name="s")
    ii = pltpu.VMEM((CH,), jnp.int32)
    ff = pltpu.VMEM((CH, OUT), jnp.float32)
    kern = pl.kernel(
        _edge_body,
        out_type=[
            jax.ShapeDtypeStruct((2, N, OUT), jnp.float32),
            jax.ShapeDtypeStruct((2, ND, OUT), jnp.float32),
        ],
        mesh=mesh,
        compiler_params=_sc_compiler_params(),
        scratch_types=[
            [pltpu.VMEM((3 * CH,), jnp.int32)] * 2,
            [ii] * 2, [ii] * 2, [ii] * 2, [ii] * 2,
            [ff] * 2, [ff] * 2, [ff] * 2, [ff] * 2, [ff] * 2,
            pltpu.VMEM((DK,), jnp.float32),
            [pltpu.SemaphoreType.DMA] * 2,
            [pltpu.SemaphoreType.DMA] * 2,
            [pltpu.SemaphoreType.DMA] * 2,
            pltpu.VMEM_SHARED((NA, OUT), jnp.float32),
            pltpu.VMEM_SHARED((ND, OUT), jnp.float32),
        ],
    )
    return kern(q_tab, kt_tab, vt_tab, packed, mvec, zv, zd)


def _update_body(pv_ref, pd_ref, x_ref, nt_ref, wa_ref, ba_ref,
                 gam_ref, bet_ref, sig_ref, pe_ref, o_ref):
    num = pv_ref[0] + pv_ref[1]
    den8 = pd_ref[0] + pd_ref[1]
    den = lax.dot_general(den8, pe_ref[...], (((1,), (0,)), ((), ())),
                          precision=lax.Precision.HIGHEST,
                          preferred_element_type=jnp.float32)
    den = jnp.where(den <= 0.0, 1.0, den)
    aggr = num / den
    g = 0.5 * aggr * (1.0 + lax.erf(aggr * (1.0 / math.sqrt(2.0))))

    nt = nt_ref[...]
    x = x_ref[...]
    trans = jnp.zeros((BN, OUT), jnp.float32)
    a = jnp.zeros((BN, 1), jnp.float32)
    gam = jnp.zeros((BN, OUT), jnp.float32)
    bet = jnp.zeros((BN, OUT), jnp.float32)
    for t in range(T):
        y = lax.dot_general(g, wa_ref[t], (((1,), (0,)), ((), ())),
                            precision=lax.Precision.HIGHEST,
                            preferred_element_type=jnp.float32)
        y = y + ba_ref[t][None, :]
        mask = nt == t
        trans = jnp.where(mask, y, trans)
        a = jnp.where(mask, sig_ref[0, t], a)
        gam = jnp.where(mask, gam_ref[t][None, :], gam)
        bet = jnp.where(mask, bet_ref[t][None, :], bet)

    res = trans * a + x * (1.0 - a)
    mu = jnp.mean(res, axis=1, keepdims=True)
    var = jnp.mean((res - mu) ** 2, axis=1, keepdims=True)
    normed = (res - mu) / jnp.sqrt(var + 1e-5)
    o_ref[...] = normed * gam + bet


def _update(pv, pd8, x, nt2, Wa, ba, gamma, beta, sig, pe):
    full = lambda s: pl.BlockSpec(s, lambda i: tuple(0 for _ in s))
    return pl.pallas_call(
        _update_body,
        grid=(NB,),
        in_specs=[
            pl.BlockSpec((2, BN, OUT), lambda i: (0, i, 0)),
            pl.BlockSpec((2, BN, H), lambda i: (0, i, 0)),
            pl.BlockSpec((BN, IN), lambda i: (i, 0)),
            pl.BlockSpec((BN, 1), lambda i: (i, 0)),
            full((T, OUT, OUT)), full((T, OUT)),
            full((T, OUT)), full((T, OUT)),
            full((1, T)), full((H, OUT)),
        ],
        out_specs=pl.BlockSpec((BN, OUT), lambda i: (i, 0)),
        out_shape=jax.ShapeDtypeStruct((N, OUT), jnp.float32),
    )(pv, pd8, x, nt2, Wa, ba, gamma, beta, sig, pe)


def kernel(meta_xs, node_type, edge_index, edge_type, Wk, bk, Wq, bq, Wv, bv,
           Wa, ba, ln_gamma, ln_beta, relation_pri, relation_att, relation_msg,
           skip):
    x = meta_xs
    nt2 = node_type.reshape(N, 1)
    src = edge_index[0]
    dst = edge_index[1]
    et = edge_type

    eyeH = jnp.eye(H, dtype=jnp.float32)
    att_s = relation_att * (relation_pri / math.sqrt(DK))[:, :, None, None]
    Batt = jnp.einsum('rhdf,hg->rhdgf', att_s, eyeH).reshape(R, OUT, OUT)
    Bmsg = jnp.einsum('rhdf,hg->rhdgf', relation_msg, eyeH).reshape(R, OUT, OUT)

    lane_head0 = jnp.arange(IN, dtype=jnp.int32) // DK
    pe2 = (lane_head0[:, None] == jnp.arange(H, dtype=jnp.int32)[None, :]
           ).astype(jnp.float32)

    q_tab, kt_tab, vt_tab, mq, mk = _prep(x, nt2, Wk, bk, Wq, bq, Wv, bv,
                                          Batt, Bmsg, pe2)
    kt_tab = kt_tab.reshape(R * N, OUT)
    vt_tab = vt_tab.reshape(R * N, OUT)

    mb = jnp.sqrt(jnp.max(mq, axis=0)[0]) * jnp.sqrt(jnp.max(mk, axis=0)[0])
    mvec = jnp.concatenate([mb, jnp.zeros((DK - H,), jnp.float32)])

    pad_n = EP - E
    pr = jnp.arange(pad_n, dtype=jnp.int32)
    src_p = jnp.concatenate([src, pr % N])
    dst_p = jnp.concatenate([dst, N + (pr % 16)])
    et_p = jnp.concatenate([et, pr % R])
    packed = jnp.stack([src_p.reshape(NW, NCHUNK, CH),
                        dst_p.reshape(NW, NCHUNK, CH),
                        et_p.reshape(NW, NCHUNK, CH)], axis=2).reshape(-1)

    zv = jnp.zeros((NA, OUT), jnp.float32)
    zd = jnp.zeros((ND, OUT), jnp.float32)
    pv, pd = _edge(q_tab, kt_tab, vt_tab, packed, mvec, zv, zd)

    pd8 = pd.reshape(2, ND * OUT // H, H)[:, :N, :]

    sig = jax.nn.sigmoid(skip).reshape(1, T)
    lane_head = jnp.arange(OUT, dtype=jnp.int32) // DK
    pe = (lane_head[None, :] == jnp.arange(H, dtype=jnp.int32)[:, None]
          ).astype(jnp.float32)

    return _update(pv, pd8, x, nt2, Wa, ba, ln_gamma, ln_beta, sig, pe)
